# baseline (speedup 1.0000x reference)
import functools

import jax
import jax.numpy as jnp
from jax import lax
from jax.experimental import pallas as pl
from jax.experimental.pallas import tpu as pltpu
from jax.experimental.pallas import tpu_sc as plsc

_LANES = 16
_NUM_CORES = 2
_NUM_SUBCORES = 16
_NUM_WORKERS = _NUM_CORES * _NUM_SUBCORES
_UNROLL = 16
_NBUF = 4
_TCHUNK = 160


def _untile_table(emb_t):
    dim, vocab = emb_t.shape
    grid = vocab // _TCHUNK

    def body(in_ref, out_ref):
        out_ref[...] = jnp.swapaxes(in_ref[...], 0, 1).reshape(_TCHUNK * dim)

    flat = pl.pallas_call(
        body,
        grid=(grid,),
        in_specs=[pl.BlockSpec((dim, _TCHUNK), lambda c: (0, c))],
        out_specs=pl.BlockSpec((_TCHUNK * dim,), lambda c: (c,)),
        out_shape=jax.ShapeDtypeStruct((vocab * dim,), jnp.float32),
    )(emb_t)
    return flat.reshape(vocab, dim)


def _sc_pool(x_t, emb_lin, batch, seq, dim):
    bpw = batch // _NUM_WORKERS
    assert bpw <= 128
    nvec = dim // _LANES
    mesh = plsc.VectorSubcoreMesh(core_axis_name="c", subcore_axis_name="s")

    @functools.partial(
        pl.kernel,
        mesh=mesh,
        compiler_params=pltpu.CompilerParams(use_tc_tiling_on_sc=False),
        out_type=jax.ShapeDtypeStruct((batch, dim), jnp.float32),
        scratch_types=[
            pltpu.VMEM((seq, bpw), jnp.int32),
            pltpu.VMEM((_NBUF, bpw, dim), jnp.float32),
            pltpu.VMEM((bpw, dim), jnp.float32),
        ] + [pltpu.SemaphoreType.DMA] * _NBUF,
    )
    def pool(xt_hbm, emb_hbm, out_hbm, idx_v, rows_v, pooled_v, *sems):
        wid = lax.axis_index("s") * _NUM_CORES + lax.axis_index("c")
        col0 = pl.multiple_of(wid * bpw, 8)
        pltpu.sync_copy(xt_hbm.at[:, pl.ds(col0, bpw)], idx_v)

        zero = jnp.zeros((_LANES,), jnp.float32)

        def zero_body(i, carry):
            for v in range(nvec):
                pooled_v[i, pl.ds(v * _LANES, _LANES)] = zero
            return carry

        lax.fori_loop(0, bpw, zero_body, 0)

        def issue(l, b):
            pltpu.async_copy(
                emb_hbm.at[idx_v.at[l]], rows_v.at[b], sems[b])

        def drain(l, b):
            pltpu.make_async_copy(
                emb_hbm.at[idx_v.at[l]], rows_v.at[b], sems[b]).wait()

        for b in range(_NBUF):
            issue(b, b)

        def do_group(g, carry):
            for b in range(_NBUF):
                l = g * _NBUF + b
                drain(l, b)

                def acc_body(i, carry2):
                    r0 = i * _UNROLL
                    for dr in range(_UNROLL):
                        for v in range(nvec):
                            val = rows_v[b, r0 + dr, pl.ds(v * _LANES, _LANES)]
                            plsc.addupdate(
                                pooled_v.at[r0 + dr, pl.ds(v * _LANES, _LANES)],
                                val)
                    return carry2

                lax.fori_loop(0, bpw // _UNROLL, acc_body, 0)

                @pl.when(l + _NBUF < seq)
                def _():
                    issue(l + _NBUF, b)

            return carry

        lax.fori_loop(0, seq // _NBUF, do_group, 0)
        for l in range(seq - seq % _NBUF, seq):
            b = l % _NBUF
            drain(l, b)

            def tail_body(i, carry2, _b=b):
                r0 = i * _UNROLL
                for dr in range(_UNROLL):
                    for v in range(nvec):
                        val = rows_v[_b, r0 + dr, pl.ds(v * _LANES, _LANES)]
                        plsc.addupdate(
                            pooled_v.at[r0 + dr, pl.ds(v * _LANES, _LANES)], val)
                return carry2

            lax.fori_loop(0, bpw // _UNROLL, tail_body, 0)

        pltpu.sync_copy(pooled_v, out_hbm.at[pl.ds(wid * bpw, bpw)])

    return pool(x_t, emb_lin)


def _mlp_head(pooled, W1, b1, W2, b2, seq):
    batch, dim = pooled.shape
    hidden = W1.shape[1]

    def body(p_ref, w1_ref, b1_ref, w2_ref, b2_ref, o_ref):
        h = p_ref[...] * (1.0 / seq)
        z = jnp.dot(h, w1_ref[...], preferred_element_type=jnp.float32)
        z = jnp.maximum(z + b1_ref[...], 0.0)
        logit = jnp.dot(z, w2_ref[...], preferred_element_type=jnp.float32)
        o_ref[...] = jax.nn.sigmoid(logit + b2_ref[...])

    out = pl.pallas_call(
        body,
        out_shape=jax.ShapeDtypeStruct((batch, 1), jnp.float32),
    )(pooled, W1, b1.reshape(1, hidden), W2, b2.reshape(1, 1))
    return out.reshape(batch)


def kernel(x, emb, W1, b1, W2, b2):
    batch, seq = x.shape
    _, dim = emb.shape
    pooled = _sc_pool(x.T, emb, batch, seq, dim)
    return _mlp_head(pooled, W1, b1, W2, b2, seq)
